# baseline (speedup 1.0000x reference)
import jax
import jax.numpy as jnp
from jax import lax
from jax.experimental import pallas as pl
from jax.experimental.pallas import tpu as pltpu
from jax.experimental.pallas import tpu_sc as plsc

BATCH = 16384
EMBED_DIM = 32
NUM_ROWS = 1000000
NUM_CORES = 2
NUM_SUBCORES = 16
NUM_WORKERS = NUM_CORES * NUM_SUBCORES
BPW = BATCH // NUM_WORKERS
LANES = 16


def _sc_body(uids_hbm, iids_hbm, uemb_hbm, iemb_hbm, ubias_hbm, ibias_hbm,
             out_hbm, uid_v, iid_v, uoffs_v, ioffs_v, ug_v, ig_v,
             ub_v, ib_v, out_v, sem):
    wid = lax.axis_index("s") * NUM_CORES + lax.axis_index("c")
    base = wid * BPW

    pltpu.sync_copy(uids_hbm.at[wid], uid_v)
    pltpu.sync_copy(iids_hbm.at[wid], iid_v)

    copies = [
        pltpu.async_copy(ubias_hbm.at[uid_v], ub_v, sem),
        pltpu.async_copy(ibias_hbm.at[iid_v], ib_v, sem),
    ]

    def offs_block(b, carry):
        sl = pl.ds(b * LANES, LANES)
        uid = uid_v[sl]
        iid = iid_v[sl]
        for c in range(EMBED_DIM):
            csl = pl.ds(c * BPW + b * LANES, LANES)
            uoffs_v[csl] = uid + c * NUM_ROWS
            ioffs_v[csl] = iid + c * NUM_ROWS
        return carry

    lax.fori_loop(0, BPW // LANES, offs_block, 0)

    for c in range(EMBED_DIM):
        csl = pl.ds(c * BPW, BPW)
        copies.append(pltpu.async_copy(uemb_hbm.at[uoffs_v.at[csl]], ug_v.at[csl], sem))
        copies.append(pltpu.async_copy(iemb_hbm.at[ioffs_v.at[csl]], ig_v.at[csl], sem))
    for cp in copies:
        cp.wait()

    def block(b, carry):
        sl = pl.ds(b * LANES, LANES)
        acc = ub_v[sl] + ib_v[sl]
        for c in range(EMBED_DIM):
            acc = acc + ug_v[pl.ds(c * BPW + b * LANES, LANES)] * \
                ig_v[pl.ds(c * BPW + b * LANES, LANES)]
        out_v[sl] = acc
        return carry

    lax.fori_loop(0, BPW // LANES, block, 0)

    pltpu.sync_copy(out_v, out_hbm.at[pl.ds(base, BPW)])


@jax.jit
def kernel(user_ids, item_ids, user_embeddings, item_embeddings,
           user_biases, item_biases):
    uids = user_ids.reshape(NUM_WORKERS, BPW).astype(jnp.int32)
    iids = item_ids.reshape(NUM_WORKERS, BPW).astype(jnp.int32)
    uflat = user_embeddings.T.reshape(-1)
    iflat = item_embeddings.T.reshape(-1)
    ubias = user_biases.reshape(-1)
    ibias = item_biases.reshape(-1)

    run = pl.kernel(
        _sc_body,
        out_type=jax.ShapeDtypeStruct((BATCH,), jnp.float32),
        mesh=plsc.VectorSubcoreMesh(
            core_axis_name="c", subcore_axis_name="s",
            num_cores=NUM_CORES, num_subcores=NUM_SUBCORES),
        compiler_params=pltpu.CompilerParams(needs_layout_passes=False),
        scratch_types=[
            pltpu.VMEM((BPW,), jnp.int32),
            pltpu.VMEM((BPW,), jnp.int32),
            pltpu.VMEM((EMBED_DIM * BPW,), jnp.int32),
            pltpu.VMEM((EMBED_DIM * BPW,), jnp.int32),
            pltpu.VMEM((EMBED_DIM * BPW,), jnp.float32),
            pltpu.VMEM((EMBED_DIM * BPW,), jnp.float32),
            pltpu.VMEM((BPW,), jnp.float32),
            pltpu.VMEM((BPW,), jnp.float32),
            pltpu.VMEM((BPW,), jnp.float32),
            pltpu.SemaphoreType.DMA,
        ],
    )
    return run(uids, iids, uflat, iflat, ubias, ibias)


# speedup vs baseline: 2.5520x; 2.5520x over previous
import functools

import jax
import jax.numpy as jnp
from jax import lax
from jax.experimental import pallas as pl
from jax.experimental.pallas import tpu as pltpu
from jax.experimental.pallas import tpu_sc as plsc

BATCH = 16384
EMBED_DIM = 32
NUM_CORES = 2
NUM_SUBCORES = 16
NUM_WORKERS = NUM_CORES * NUM_SUBCORES
BPW = BATCH // NUM_WORKERS
IDX_CHUNK = 128
NCHUNK = BPW // IDX_CHUNK
LANES = 16


def _sc_body(uids_hbm, iids_hbm, uemb_hbm, iemb_hbm, ubias_hbm, ibias_hbm,
             out_hbm, uid_v, iid_v, urows_v, irows_v, ub_v, ib_v, out_v, sem):
    wid = lax.axis_index("s") * NUM_CORES + lax.axis_index("c")
    base = wid * BPW

    pltpu.sync_copy(uids_hbm.at[wid], uid_v)
    pltpu.sync_copy(iids_hbm.at[wid], iid_v)

    copies = []
    for k in range(NCHUNK):
        rows = pl.ds(k * IDX_CHUNK, IDX_CHUNK)
        copies.append(pltpu.async_copy(uemb_hbm.at[uid_v.at[k]], urows_v.at[rows], sem))
        copies.append(pltpu.async_copy(iemb_hbm.at[iid_v.at[k]], irows_v.at[rows], sem))
        copies.append(pltpu.async_copy(ubias_hbm.at[uid_v.at[k]], ub_v.at[rows], sem))
        copies.append(pltpu.async_copy(ibias_hbm.at[iid_v.at[k]], ib_v.at[rows], sem))
    for cp in copies:
        cp.wait()

    lane = lax.iota(jnp.int32, LANES)

    def block(b, carry):
        r0 = b * LANES
        row_idx = r0 + lane
        acc = ub_v[pl.ds(r0, LANES)] + ib_v[pl.ds(r0, LANES)]
        for c2 in range(EMBED_DIM // 2):
            col = jnp.full((LANES,), c2, jnp.int32)
            u_pair = plsc.load_gather(urows_v, [row_idx, col])
            i_pair = plsc.load_gather(irows_v, [row_idx, col])
            ua, ub = plsc.unpack(plsc.bitcast(u_pair, jnp.bfloat16),
                                 format=plsc.PackFormat.INTERLEAVED)
            ia, ib = plsc.unpack(plsc.bitcast(i_pair, jnp.bfloat16),
                                 format=plsc.PackFormat.INTERLEAVED)
            acc = acc + ua * ia + ub * ib
        out_v[pl.ds(r0, LANES)] = acc
        return carry

    lax.fori_loop(0, BPW // LANES, block, 0)

    pltpu.sync_copy(out_v, out_hbm.at[pl.ds(base, BPW)])


@functools.partial(jax.jit, static_argnums=())
def kernel(user_ids, item_ids, user_embeddings, item_embeddings,
           user_biases, item_biases):
    uids = user_ids.reshape(NUM_WORKERS, NCHUNK, IDX_CHUNK).astype(jnp.int32)
    iids = item_ids.reshape(NUM_WORKERS, NCHUNK, IDX_CHUNK).astype(jnp.int32)
    uemb16 = jax.lax.bitcast_convert_type(
        user_embeddings.astype(jnp.bfloat16).reshape(-1, EMBED_DIM // 2, 2),
        jnp.int32)
    iemb16 = jax.lax.bitcast_convert_type(
        item_embeddings.astype(jnp.bfloat16).reshape(-1, EMBED_DIM // 2, 2),
        jnp.int32)
    ubias = user_biases.reshape(-1)
    ibias = item_biases.reshape(-1)

    run = pl.kernel(
        _sc_body,
        out_type=jax.ShapeDtypeStruct((BATCH,), jnp.float32),
        compiler_params=pltpu.CompilerParams(
            needs_layout_passes=False, use_tc_tiling_on_sc=False),
        mesh=plsc.VectorSubcoreMesh(
            core_axis_name="c", subcore_axis_name="s",
            num_cores=NUM_CORES, num_subcores=NUM_SUBCORES),
        scratch_types=[
            pltpu.VMEM((NCHUNK, IDX_CHUNK), jnp.int32),
            pltpu.VMEM((NCHUNK, IDX_CHUNK), jnp.int32),
            pltpu.VMEM((BPW, EMBED_DIM // 2), jnp.int32),
            pltpu.VMEM((BPW, EMBED_DIM // 2), jnp.int32),
            pltpu.VMEM((BPW,), jnp.float32),
            pltpu.VMEM((BPW,), jnp.float32),
            pltpu.VMEM((BPW,), jnp.float32),
            pltpu.SemaphoreType.DMA,
        ],
    )
    return run(uids, iids, uemb16, iemb16, ubias, ibias)


# speedup vs baseline: 5.7199x; 2.2414x over previous
import functools

import jax
import jax.numpy as jnp
from jax import lax
from jax.experimental import pallas as pl
from jax.experimental.pallas import tpu as pltpu
from jax.experimental.pallas import tpu_sc as plsc

BATCH = 16384
EMBED_DIM = 32
NUM_CORES = 2
NUM_SUBCORES = 16
NUM_WORKERS = NUM_CORES * NUM_SUBCORES
BPW = BATCH // NUM_WORKERS
IDX_CHUNK = 128
NCHUNK = BPW // IDX_CHUNK
LANES = 16


def _sc_body(uids_hbm, iids_hbm, uemb_hbm, iemb_hbm, ubias_hbm, ibias_hbm,
             out_hbm, uid_v, iid_v, urows_v, irows_v, ub_v, ib_v, out_v, sem):
    wid = lax.axis_index("s") * NUM_CORES + lax.axis_index("c")
    base = wid * BPW

    pltpu.sync_copy(uids_hbm.at[wid], uid_v)
    pltpu.sync_copy(iids_hbm.at[wid], iid_v)

    copies = []
    for k in range(NCHUNK):
        rows = pl.ds(k * IDX_CHUNK, IDX_CHUNK)
        copies.append(pltpu.async_copy(uemb_hbm.at[uid_v.at[k]], urows_v.at[rows], sem))
        copies.append(pltpu.async_copy(iemb_hbm.at[iid_v.at[k]], irows_v.at[rows], sem))
        copies.append(pltpu.async_copy(ubias_hbm.at[uid_v.at[k]], ub_v.at[rows], sem))
        copies.append(pltpu.async_copy(ibias_hbm.at[iid_v.at[k]], ib_v.at[rows], sem))
    for cp in copies:
        cp.wait()

    lane = lax.iota(jnp.int32, LANES)

    def block(b, carry):
        r0 = b * LANES
        row_idx = r0 + lane
        acc = ub_v[pl.ds(r0, LANES)] + ib_v[pl.ds(r0, LANES)]
        for d in range(EMBED_DIM):
            col = jnp.full((LANES,), d, jnp.int32)
            u = plsc.load_gather(urows_v, [row_idx, col])
            i = plsc.load_gather(irows_v, [row_idx, col])
            acc = acc + u * i
        out_v[pl.ds(r0, LANES)] = acc
        return carry

    lax.fori_loop(0, BPW // LANES, block, 0)

    pltpu.sync_copy(out_v, out_hbm.at[pl.ds(base, BPW)])


@functools.partial(jax.jit, static_argnums=())
def kernel(user_ids, item_ids, user_embeddings, item_embeddings,
           user_biases, item_biases):
    uids = user_ids.reshape(NUM_WORKERS, NCHUNK, IDX_CHUNK).astype(jnp.int32)
    iids = item_ids.reshape(NUM_WORKERS, NCHUNK, IDX_CHUNK).astype(jnp.int32)
    ubias = user_biases.reshape(-1)
    ibias = item_biases.reshape(-1)

    run = pl.kernel(
        _sc_body,
        out_type=jax.ShapeDtypeStruct((BATCH,), jnp.float32),
        compiler_params=pltpu.CompilerParams(
            needs_layout_passes=False, use_tc_tiling_on_sc=False),
        mesh=plsc.VectorSubcoreMesh(
            core_axis_name="c", subcore_axis_name="s",
            num_cores=NUM_CORES, num_subcores=NUM_SUBCORES),
        scratch_types=[
            pltpu.VMEM((NCHUNK, IDX_CHUNK), jnp.int32),
            pltpu.VMEM((NCHUNK, IDX_CHUNK), jnp.int32),
            pltpu.VMEM((BPW, EMBED_DIM), jnp.float32),
            pltpu.VMEM((BPW, EMBED_DIM), jnp.float32),
            pltpu.VMEM((BPW,), jnp.float32),
            pltpu.VMEM((BPW,), jnp.float32),
            pltpu.VMEM((BPW,), jnp.float32),
            pltpu.SemaphoreType.DMA,
        ],
    )
    return run(uids, iids, user_embeddings, item_embeddings, ubias, ibias)
